# Initial kernel scaffold; baseline (speedup 1.0000x reference)
import jax
import jax.numpy as jnp
from jax.experimental import pallas as pl


def kernel(input, other, coupling, W_q, W_kv, W_out, b_out):
    raise NotImplementedError("write your pallas kernel here")


# speedup vs baseline: 16.9647x; 16.9647x over previous
import functools

import jax
import jax.numpy as jnp
from jax import lax
from jax.experimental import pallas as pl
from jax.experimental.pallas import tpu as pltpu
from jax.experimental.pallas import tpu_sc as plsc

N_NODES = 10000
N_EDGES = 320000
D_MODEL = 128
N_HEADS = 8
HEAD_DIM = 16
ROW_W = 144
N_WORKERS = 32
EDGES_PER_W = N_EDGES // N_WORKERS
BLK = 40
N_BLKS = EDGES_PER_W // BLK
ACC_ROWS = N_NODES
ROWS_PER_TILE = ACC_ROWS // 16


def _proj_body(x_ref, w_ref, o_ref):
    o_ref[...] = lax.dot_general(
        x_ref[...], w_ref[...], (((1,), (1,)), ((), ())),
        preferred_element_type=jnp.float32)


def _project(x, w, blk_rows=400):
    n, d_in = x.shape
    d_out = w.shape[0]
    grid = (n // blk_rows,)
    return pl.pallas_call(
        _proj_body,
        grid=grid,
        in_specs=[
            pl.BlockSpec((blk_rows, d_in), lambda i: (i, 0)),
            pl.BlockSpec((d_out, d_in), lambda i: (0, 0)),
        ],
        out_specs=pl.BlockSpec((blk_rows, d_out), lambda i: (i, 0)),
        out_shape=jax.ShapeDtypeStruct((n, d_out), jnp.float32),
    )(x, w)


def _edge_kernel_body(q_hbm, kv_hbm, t_hbm, s_hbm, zeros_hbm, out_hbm,
                      tidx, sidx, qrows, kvrows, numbuf, acc, sem_q, sem_kv):
    cid = lax.axis_index("c")
    sid = lax.axis_index("s")
    wid = sid * 2 + cid

    pltpu.sync_copy(zeros_hbm, acc.at[pl.ds(sid * ROWS_PER_TILE, ROWS_PER_TILE)])
    plsc.subcore_barrier()

    lane = lax.iota(jnp.int32, 16)
    perms = [lane ^ st for st in (8, 4, 2, 1)]

    def block_body(blk, _):
        base = wid * EDGES_PER_W + blk * BLK
        pltpu.sync_copy(t_hbm.at[pl.ds(base, BLK)], tidx)
        pltpu.sync_copy(s_hbm.at[pl.ds(base, BLK)], sidx)
        cp_q = pltpu.async_copy(q_hbm.at[tidx], qrows, sem_q)
        cp_kv = pltpu.async_copy(kv_hbm.at[sidx], kvrows, sem_kv)
        cp_q.wait()
        cp_kv.wait()

        def edge_body(i, carry):
            dv = jnp.zeros((16,), jnp.float32)
            for h in range(N_HEADS):
                qh = qrows[i, pl.ds(HEAD_DIM * h, HEAD_DIM)]
                kh = kvrows[i, pl.ds(HEAD_DIM * h, HEAD_DIM)]
                prod = qh * kh
                for p in perms:
                    prod = prod + jnp.take(prod, p)
                exv = jnp.exp(prod * 0.25)
                vh = kvrows[i, pl.ds(D_MODEL + HEAD_DIM * h, HEAD_DIM)]
                numbuf[i, pl.ds(HEAD_DIM * h, HEAD_DIM)] = exv * vh
                dv = dv + jnp.where(lane == h, exv, 0.0)
            numbuf[i, pl.ds(D_MODEL, HEAD_DIM)] = dv
            return carry

        lax.fori_loop(0, BLK, edge_body, 0)
        pltpu.sync_copy(numbuf, acc.at[tidx], add=True)
        return _

    lax.fori_loop(0, N_BLKS, block_body, 0)

    plsc.subcore_barrier()
    pltpu.sync_copy(acc.at[pl.ds(sid * ROWS_PER_TILE, ROWS_PER_TILE)],
                    out_hbm.at[cid, pl.ds(sid * ROWS_PER_TILE, ROWS_PER_TILE)])


def _edge_stage(q, kv, t, s, zeros):
    mesh = plsc.VectorSubcoreMesh(core_axis_name="c", subcore_axis_name="s")
    kern = pl.kernel(
        _edge_kernel_body,
        out_type=jax.ShapeDtypeStruct((2, ACC_ROWS, ROW_W), jnp.float32),
        mesh=mesh,
        scratch_types=[
            pltpu.VMEM((BLK,), jnp.int32),
            pltpu.VMEM((BLK,), jnp.int32),
            pltpu.VMEM((BLK, D_MODEL), jnp.float32),
            pltpu.VMEM((BLK, 2 * D_MODEL), jnp.float32),
            pltpu.VMEM((BLK, ROW_W), jnp.float32),
            pltpu.VMEM_SHARED((ACC_ROWS, ROW_W), jnp.float32),
            pltpu.SemaphoreType.DMA,
            pltpu.SemaphoreType.DMA,
        ],
        compiler_params=pltpu.CompilerParams(use_tc_tiling_on_sc=False),
    )
    return kern(q, kv, t, s, zeros)


def _epi_body(a0_ref, a1_ref, e_ref, w_ref, b_ref, o_ref):
    a = a0_ref[...] + a1_ref[...]
    num = a[:, :D_MODEL]
    den = a[:, D_MODEL:D_MODEL + N_HEADS]
    denf = lax.dot_general(den, e_ref[...], (((1,), (0,)), ((), ())),
                           preferred_element_type=jnp.float32)
    y = num / (denf + 1e-16)
    o_ref[...] = lax.dot_general(y, w_ref[...], (((1,), (1,)), ((), ())),
                                 preferred_element_type=jnp.float32) + b_ref[...]


def _epilogue(acc, w_out, b_out, blk_rows=400):
    expand = jnp.repeat(jnp.eye(N_HEADS, dtype=jnp.float32), HEAD_DIM, axis=1)
    grid = (N_NODES // blk_rows,)
    return pl.pallas_call(
        _epi_body,
        grid=grid,
        in_specs=[
            pl.BlockSpec((blk_rows, ROW_W), lambda i: (i, 0)),
            pl.BlockSpec((blk_rows, ROW_W), lambda i: (i, 0)),
            pl.BlockSpec((N_HEADS, D_MODEL), lambda i: (0, 0)),
            pl.BlockSpec((D_MODEL, D_MODEL), lambda i: (0, 0)),
            pl.BlockSpec((1, D_MODEL), lambda i: (0, 0)),
        ],
        out_specs=pl.BlockSpec((blk_rows, D_MODEL), lambda i: (i, 0)),
        out_shape=jax.ShapeDtypeStruct((N_NODES, D_MODEL), jnp.float32),
    )(acc[0], acc[1], expand, w_out, b_out.reshape(1, D_MODEL))


def kernel(input, other, coupling, W_q, W_kv, W_out, b_out):
    q = _project(input, W_q)
    kv = _project(other, W_kv)
    t = coupling[0]
    s = coupling[1]
    zeros = jnp.zeros((ROWS_PER_TILE, ROW_W), jnp.float32)
    acc = _edge_stage(q, kv, t, s, zeros)
    return _epilogue(acc[:, :N_NODES], W_out, b_out)


# speedup vs baseline: 20.7038x; 1.2204x over previous
import functools

import jax
import jax.numpy as jnp
from jax import lax
from jax.experimental import pallas as pl
from jax.experimental.pallas import tpu as pltpu
from jax.experimental.pallas import tpu_sc as plsc

N_NODES = 10000
N_EDGES = 320000
D_MODEL = 128
N_HEADS = 8
HEAD_DIM = 16
ROW_W = 144
N_WORKERS = 32
EDGES_PER_W = N_EDGES // N_WORKERS
BLK = 40
N_BLKS = EDGES_PER_W // BLK
IDX_W = 20
IDX_H = IDX_W // 2
N_PAIRS = N_BLKS // 2
ACC_ROWS = N_NODES
ROWS_PER_TILE = ACC_ROWS // 16


def _proj_body(x_ref, w_ref, o_ref):
    o_ref[...] = lax.dot_general(
        x_ref[...], w_ref[...], (((1,), (1,)), ((), ())),
        preferred_element_type=jnp.float32)


def _project(x, w, blk_rows=400):
    n, d_in = x.shape
    d_out = w.shape[0]
    grid = (n // blk_rows,)
    return pl.pallas_call(
        _proj_body,
        grid=grid,
        in_specs=[
            pl.BlockSpec((blk_rows, d_in), lambda i: (i, 0)),
            pl.BlockSpec((d_out, d_in), lambda i: (0, 0)),
        ],
        out_specs=pl.BlockSpec((blk_rows, d_out), lambda i: (i, 0)),
        out_shape=jax.ShapeDtypeStruct((n, d_out), jnp.float32),
    )(x, w)


def _edge_kernel_body(q_hbm, kv_hbm, t_hbm, s_hbm, zeros_hbm, out_hbm,
                      tidx, sidx,
                      qrows_a, kvrows_a, qrows_b, kvrows_b,
                      numbuf, acc, sq_a, sk_a, sq_b, sk_b):
    cid = lax.axis_index("c")
    sid = lax.axis_index("s")
    wid = sid * 2 + cid
    ebase = wid * EDGES_PER_W

    pltpu.sync_copy(zeros_hbm, acc.at[pl.ds(sid * ROWS_PER_TILE, ROWS_PER_TILE)])
    plsc.subcore_barrier()

    lane = lax.iota(jnp.int32, 16)
    perms = [lane ^ st for st in (8, 4, 2, 1)]

    bbase = wid * N_BLKS

    def refresh_idx(start_blk, row0, nrows):
        pltpu.sync_copy(t_hbm.at[pl.ds(bbase + start_blk, nrows)],
                        tidx.at[pl.ds(row0, nrows)])
        pltpu.sync_copy(s_hbm.at[pl.ds(bbase + start_blk, nrows)],
                        sidx.at[pl.ds(row0, nrows)])

    def issue(r, qrows, kvrows, sq, sk):
        pltpu.async_copy(q_hbm.at[tidx.at[r]], qrows, sq)
        pltpu.async_copy(kv_hbm.at[sidx.at[r]], kvrows, sk)

    def wait(r, qrows, kvrows, sq, sk):
        pltpu.make_async_copy(q_hbm.at[tidx.at[r]], qrows, sq).wait()
        pltpu.make_async_copy(kv_hbm.at[sidx.at[r]], kvrows, sk).wait()

    def compute(r, qrows, kvrows):
        def edge_body(i, carry):
            dv = jnp.zeros((16,), jnp.float32)
            for h in range(N_HEADS):
                qh = qrows[i, pl.ds(HEAD_DIM * h, HEAD_DIM)]
                kh = kvrows[i, pl.ds(HEAD_DIM * h, HEAD_DIM)]
                prod = qh * kh
                for p in perms:
                    prod = prod + jnp.take(prod, p)
                exv = jnp.exp(prod * 0.25)
                vh = kvrows[i, pl.ds(D_MODEL + HEAD_DIM * h, HEAD_DIM)]
                numbuf[i, pl.ds(HEAD_DIM * h, HEAD_DIM)] = exv * vh
                dv = dv + jnp.where(lane == h, exv, 0.0)
            numbuf[i, pl.ds(D_MODEL, HEAD_DIM)] = dv
            return carry

        lax.fori_loop(0, BLK, edge_body, 0)
        pltpu.sync_copy(numbuf, acc.at[tidx.at[r]], add=True)

    refresh_idx(0, 0, IDX_H)
    issue(0, qrows_a, kvrows_a, sq_a, sk_a)

    def body(it, _):
        blk_a = 2 * it
        ra = lax.rem(blk_a, IDX_W)
        rb = ra + 1
        wait(ra, qrows_a, kvrows_a, sq_a, sk_a)
        issue(rb, qrows_b, kvrows_b, sq_b, sk_b)
        compute(ra, qrows_a, kvrows_a)
        wait(rb, qrows_b, kvrows_b, sq_b, sk_b)

        @pl.when((lax.rem(ra, IDX_H) == 2) & (blk_a + 8 + IDX_H <= N_BLKS))
        def _refresh():
            row0 = jnp.where(ra == 2, IDX_H, 0).astype(jnp.int32)
            refresh_idx(blk_a + 8, row0, IDX_H)

        @pl.when(it < N_PAIRS - 1)
        def _issue_next():
            issue(lax.rem(blk_a + 2, IDX_W), qrows_a, kvrows_a, sq_a, sk_a)

        compute(rb, qrows_b, kvrows_b)
        return _

    lax.fori_loop(0, N_PAIRS, body, 0)

    plsc.subcore_barrier()
    pltpu.sync_copy(acc.at[pl.ds(sid * ROWS_PER_TILE, ROWS_PER_TILE)],
                    out_hbm.at[cid, pl.ds(sid * ROWS_PER_TILE, ROWS_PER_TILE)])


def _edge_stage(q, kv, t, s, zeros):
    mesh = plsc.VectorSubcoreMesh(core_axis_name="c", subcore_axis_name="s")
    kern = pl.kernel(
        _edge_kernel_body,
        out_type=jax.ShapeDtypeStruct((2, N_NODES, ROW_W), jnp.float32),
        mesh=mesh,
        scratch_types=[
            pltpu.VMEM((IDX_W, BLK), jnp.int32),
            pltpu.VMEM((IDX_W, BLK), jnp.int32),
            pltpu.VMEM((BLK, D_MODEL), jnp.float32),
            pltpu.VMEM((BLK, 2 * D_MODEL), jnp.float32),
            pltpu.VMEM((BLK, D_MODEL), jnp.float32),
            pltpu.VMEM((BLK, 2 * D_MODEL), jnp.float32),
            pltpu.VMEM((BLK, ROW_W), jnp.float32),
            pltpu.VMEM_SHARED((N_NODES, ROW_W), jnp.float32),
            pltpu.SemaphoreType.DMA,
            pltpu.SemaphoreType.DMA,
            pltpu.SemaphoreType.DMA,
            pltpu.SemaphoreType.DMA,
        ],
        compiler_params=pltpu.CompilerParams(use_tc_tiling_on_sc=False),
    )
    return kern(q, kv, t, s, zeros)


def _epi_body(a0_ref, a1_ref, e_ref, w_ref, b_ref, o_ref):
    a = a0_ref[...] + a1_ref[...]
    num = a[:, :D_MODEL]
    den = a[:, D_MODEL:D_MODEL + N_HEADS]
    denf = lax.dot_general(den, e_ref[...], (((1,), (0,)), ((), ())),
                           preferred_element_type=jnp.float32)
    y = num / (denf + 1e-16)
    o_ref[...] = lax.dot_general(y, w_ref[...], (((1,), (1,)), ((), ())),
                                 preferred_element_type=jnp.float32) + b_ref[...]


def _epilogue(acc, w_out, b_out, blk_rows=400):
    expand = jnp.repeat(jnp.eye(N_HEADS, dtype=jnp.float32), HEAD_DIM, axis=1)
    grid = (N_NODES // blk_rows,)
    return pl.pallas_call(
        _epi_body,
        grid=grid,
        in_specs=[
            pl.BlockSpec((blk_rows, ROW_W), lambda i: (i, 0)),
            pl.BlockSpec((blk_rows, ROW_W), lambda i: (i, 0)),
            pl.BlockSpec((N_HEADS, D_MODEL), lambda i: (0, 0)),
            pl.BlockSpec((D_MODEL, D_MODEL), lambda i: (0, 0)),
            pl.BlockSpec((1, D_MODEL), lambda i: (0, 0)),
        ],
        out_specs=pl.BlockSpec((blk_rows, D_MODEL), lambda i: (i, 0)),
        out_shape=jax.ShapeDtypeStruct((N_NODES, D_MODEL), jnp.float32),
    )(acc[0], acc[1], expand, w_out, b_out.reshape(1, D_MODEL))


def kernel(input, other, coupling, W_q, W_kv, W_out, b_out):
    q = _project(input, W_q)
    kv = _project(other, W_kv)
    t = coupling[0].reshape(N_EDGES // BLK, BLK)
    s = coupling[1].reshape(N_EDGES // BLK, BLK)
    zeros = jnp.zeros((ROWS_PER_TILE, ROW_W), jnp.float32)
    acc = _edge_stage(q, kv, t, s, zeros)
    return _epilogue(acc[:, :N_NODES], W_out, b_out)
